# Initial kernel scaffold; baseline (speedup 1.0000x reference)
import jax
import jax.numpy as jnp
from jax.experimental import pallas as pl


def kernel(x_paper, x_author, edge_index_cites, edge_index_writes, edge_index_rev_writes, x_covs, batch_num, is_sparsed, class_batch_info, params):
    raise NotImplementedError("write your pallas kernel here")


# speedup vs baseline: 3.4100x; 3.4100x over previous
import functools

import jax
import jax.numpy as jnp
from jax import lax
from jax.experimental import pallas as pl
from jax.experimental.pallas import tpu as pltpu
from jax.experimental.pallas import tpu_sc as plsc

N = 50000
D = 128
E = 200000
B = 64
L = 16
NC = 2
NS = 16
NW = NC * NS

NEG = -1e30

E3 = 3 * E
TPB1 = 18816
E3P = TPB1 * NW
NB1 = TPB1 // 128

CHUNK = 12800
NCH = 4
NPAD = CHUNK * NCH

RPT = 1568
NPOOL = RPT * NW
PBLK = 32
NPB = RPT // PBLK

def _pad1(x, n, val):
    return jnp.concatenate([x, jnp.full((n - x.shape[0],), val, x.dtype)])


def _lane_sum(x):
    iota = lax.iota(jnp.int32, L)
    for k in (1, 2, 4, 8):
        x = x + x.at[iota ^ k].get(mode="promise_in_bounds")
    return x


@functools.cache
def _build_sddmm():
  _mesh = plsc.VectorSubcoreMesh(core_axis_name="c", subcore_axis_name="s")

  @functools.partial(
    pl.kernel,
    out_type=(
        jax.ShapeDtypeStruct((E3P,), jnp.float32),
        jax.ShapeDtypeStruct((2 * NW, L), jnp.float32),
    ),
    mesh=_mesh,
    scratch_types=[
        pltpu.VMEM((128,), jnp.int32),
        pltpu.VMEM((128,), jnp.int32),
        pltpu.VMEM((128, D), jnp.float32),
        pltpu.VMEM((128, D), jnp.float32),
        pltpu.VMEM((128,), jnp.float32),
        pltpu.VMEM((L,), jnp.float32),
        pltpu.VMEM((L,), jnp.float32),
        pltpu.SemaphoreType.DMA,
        pltpu.SemaphoreType.DMA,
    ],
  )
  def _sddmm(qcat, kcat, srck, dstq, lgt_out, max_out,
             src_v, dst_v, krows, qrows, lgt_v, mb0, mb1, sem1, sem2):
    c = lax.axis_index("c")
    s = lax.axis_index("s")
    wid = s * NC + c
    base = wid * TPB1
    iota = lax.iota(jnp.int32, L)
    negv = jnp.full((L,), NEG, jnp.float32)

    def blk_body(b, carry):
        maxp, maxa = carry
        off = base + b * 128
        pltpu.sync_copy(srck.at[pl.ds(off, 128)], src_v)
        pltpu.sync_copy(dstq.at[pl.ds(off, 128)], dst_v)
        cp1 = pltpu.async_copy(kcat.at[src_v], krows, sem1)
        cp2 = pltpu.async_copy(qcat.at[dst_v], qrows, sem2)
        cp1.wait()
        cp2.wait()
        for g in range(8):
            def d_body(j, accs):
                dj = j * 16
                return tuple(
                    accs[e] + qrows[g * 16 + e, pl.ds(dj, 16)]
                    * krows[g * 16 + e, pl.ds(dj, 16)]
                    for e in range(16)
                )

            zero = jnp.zeros((L,), jnp.float32)
            accs = lax.fori_loop(0, 8, d_body, (zero,) * 16)
            lgt = zero
            for e in range(16):
                s = _lane_sum(accs[e])
                lgt = jnp.where(iota == e, s, lgt)
            ids = off + g * 16 + iota
            lgt = jnp.where(ids < E3, lgt, negv)
            dst16 = dst_v[pl.ds(g * 16, 16)]
            isp = dst16 < N
            maxp = jnp.maximum(maxp, jnp.where(isp, lgt, negv))
            maxa = jnp.maximum(maxa, jnp.where(isp, negv, lgt))
            lgt_v[pl.ds(g * 16, 16)] = lgt
        pltpu.sync_copy(lgt_v, lgt_out.at[pl.ds(off, 128)])
        return maxp, maxa

    mp, ma = lax.fori_loop(0, NB1, blk_body, (negv, negv))
    mb0[...] = mp
    mb1[...] = ma
    pltpu.sync_copy(mb0, max_out.at[wid])
    pltpu.sync_copy(mb1, max_out.at[NW + wid])

  return _sddmm


NPS = 50176
RPS = NPS // NS
NPASS = 8
DP = D // NPASS


@functools.cache
def _make_agg(TPT, VT):
    _mesh = plsc.VectorSubcoreMesh(core_axis_name="c", subcore_axis_name="s")
    NBLK = TPT // 128

    @functools.partial(
        pl.kernel,
        out_type=(
            jax.ShapeDtypeStruct((NPASS * NC * NPS, DP), jnp.float32),
            jax.ShapeDtypeStruct((NC * NPS,), jnp.float32),
        ),
        mesh=_mesh,
        compiler_params=pltpu.CompilerParams(use_tc_tiling_on_sc=False),
        scratch_types=[
            pltpu.VMEM((TPT,), jnp.int32),
            pltpu.VMEM((TPT,), jnp.int32),
            pltpu.VMEM((TPT,), jnp.float32),
            pltpu.VMEM((128,), jnp.int32),
            pltpu.VMEM((128,), jnp.int32),
            pltpu.VMEM((128,), jnp.float32),
            pltpu.VMEM((128, DP), jnp.float32),
            pltpu.VMEM((L,), jnp.float32),
            pltpu.VMEM((392, DP), jnp.float32),
            pltpu.VMEM((392, DP), jnp.float32),
            pltpu.VMEM((RPS,), jnp.float32),
            pltpu.VMEM((RPS,), jnp.float32),
            pltpu.VMEM_SHARED((NPS, DP), jnp.float32),
            pltpu.VMEM_SHARED((NPS,), jnp.float32),
            pltpu.SemaphoreType.DMA,
        ],
    )
    def _agg(vcat4, srcv, dstv, lgtv, mvec, agg_out, den_out,
             src_sh, dst_sh, lgt_sh, srcblk, idxblk, exblk, rows, mv,
             zrows, stg, zden, stgd, sp_agg, sp_den, sem):
        c = lax.axis_index("c")
        s = lax.axis_index("s")
        wid = s * NC + c
        base = wid * TPT
        z16 = jnp.zeros((L,), jnp.float32)

        pltpu.sync_copy(srcv.at[pl.ds(base, TPT)], src_sh)
        pltpu.sync_copy(dstv.at[pl.ds(base, TPT)], dst_sh)
        pltpu.sync_copy(lgtv.at[pl.ds(base, TPT)], lgt_sh)
        pltpu.sync_copy(mvec, mv)
        M = mv[...]

        def zfill_rows(i, _):
            for j in range(DP // 16):
                zrows[i, pl.ds(j * 16, 16)] = z16
            return 0

        lax.fori_loop(0, 392, zfill_rows, 0)

        def zfill_den(i, _):
            zden[pl.ds(i * 16, 16)] = z16
            return 0

        lax.fori_loop(0, RPS // 16, zfill_den, 0)

        def pass_body(p, _p):
            poff = p * VT
            for r in range(8):
                pltpu.sync_copy(zrows, sp_agg.at[pl.ds(s * RPS + r * 392, 392)])

            @pl.when(p == 0)
            def _():
                pltpu.sync_copy(zden, sp_den.at[pl.ds(s * RPS, RPS)])

            plsc.subcore_barrier()

            def blk(b, _b):
                boff = b * 128
                for g in range(8):
                    sl = pl.ds(boff + g * 16, 16)
                    gsl = pl.ds(g * 16, 16)
                    idxblk[gsl] = dst_sh[sl]
                    srcblk[gsl] = src_sh[sl] + poff
                    exblk[gsl] = jnp.exp(lgt_sh[sl] - M)
                pltpu.async_copy(vcat4.at[srcblk], rows, sem).wait()
                for g in range(8):
                    ex = exblk[pl.ds(g * 16, 16)]
                    for e in range(16):
                        r = g * 16 + e
                        exs = ex[e]
                        for j in range(DP // 16):
                            jsl = pl.ds(j * 16, 16)
                            rows[r, jsl] = rows[r, jsl] * exs
                pltpu.sync_copy(rows, sp_agg.at[idxblk], add=True)

                @pl.when(p == 0)
                def _():
                    pltpu.sync_copy(exblk, sp_den.at[idxblk], add=True)

                return 0

            lax.fori_loop(0, NBLK, blk, 0)
            plsc.subcore_barrier()
            obase = p * (NC * NPS) + c * NPS + s * RPS
            for r in range(8):
                pltpu.sync_copy(sp_agg.at[pl.ds(s * RPS + r * 392, 392)], stg)
                pltpu.sync_copy(stg, agg_out.at[pl.ds(obase + r * 392, 392)])

            @pl.when(p == 0)
            def _():
                pltpu.sync_copy(sp_den.at[pl.ds(s * RPS, RPS)], stgd)
                pltpu.sync_copy(stgd, den_out.at[pl.ds(c * NPS + s * RPS, RPS)])

            plsc.subcore_barrier()
            return 0

        lax.fori_loop(0, NPASS, pass_body, 0)

    return _agg


TPT_P = 12544
TPT_A = 6272
EPP = TPT_P * NW
EAP = TPT_A * NW


@functools.cache
def _build_pool():
  _mesh = plsc.VectorSubcoreMesh(core_axis_name="c", subcore_axis_name="s")

  @functools.partial(
    pl.kernel,
    out_type=jax.ShapeDtypeStruct((NW * B, D), jnp.float32),
    mesh=_mesh,
    scratch_types=[
        pltpu.VMEM((B, D), jnp.float32),
        pltpu.VMEM((PBLK, D), jnp.float32),
        pltpu.VMEM((PBLK + 16,), jnp.int32),
    ],
  )
  def _pool(x, cbi, out, acc, rbuf, cbuf):
    c = lax.axis_index("c")
    s = lax.axis_index("s")
    wid = s * NC + c
    base = wid * RPT
    z16 = jnp.zeros((L,), jnp.float32)

    def zf(i, _):
        for j in range(8):
            acc[i, pl.ds(j * 16, 16)] = z16
        return 0

    lax.fori_loop(0, B, zf, 0)

    def blk(b, _):
        off = base + b * PBLK
        pltpu.sync_copy(x.at[pl.ds(off, PBLK)], rbuf)
        pltpu.sync_copy(cbi.at[pl.ds(off, PBLK)], cbuf.at[pl.ds(0, PBLK)])

        def row(r, _2):
            cb = cbuf[pl.ds(r, 16)][0]
            for j in range(8):
                acc[cb, pl.ds(j * 16, 16)] = (acc[cb, pl.ds(j * 16, 16)]
                                              + rbuf[r, pl.ds(j * 16, 16)])
            return 0

        lax.fori_loop(0, PBLK, row, 0)
        return 0

    lax.fori_loop(0, NPB, blk, 0)
    pltpu.sync_copy(acc, out.at[pl.ds(wid * B, B)])

  return _pool


def _lin(x, wb):
    return x @ wb[0] + wb[1]


def _hgt_layer_sc(xd, edges, lp):
    sq = jnp.sqrt(jnp.float32(D))
    kk = {t: _lin(xd[t], lp['k'][t]) for t in ('paper', 'author')}
    qq = {t: _lin(xd[t], lp['q'][t]) for t in ('paper', 'author')}
    vv = {t: _lin(xd[t], lp['v'][t]) for t in ('paper', 'author')}

    k_c = (kk['paper'] @ lp['a_rel']['cites']) * (lp['p_rel']['cites'] / sq)
    k_w = (kk['author'] @ lp['a_rel']['writes']) * (lp['p_rel']['writes'] / sq)
    k_r = (kk['paper'] @ lp['a_rel']['rev_writes']) * (lp['p_rel']['rev_writes'] / sq)
    v_c = vv['paper'] @ lp['m_rel']['cites']
    v_w = vv['author'] @ lp['m_rel']['writes']
    v_r = vv['paper'] @ lp['m_rel']['rev_writes']

    qcat = jnp.concatenate([qq['paper'], qq['author']])
    kcat = jnp.concatenate([k_c, k_w, k_r])

    src_c, dst_c = edges['cites'][0], edges['cites'][1]
    src_w, dst_w = edges['writes'][0], edges['writes'][1]
    src_r, dst_r = edges['rev_writes'][0], edges['rev_writes'][1]

    srck = _pad1(jnp.concatenate([src_c, src_w + N, src_r + 2 * N]), E3P, 0)
    dstq = _pad1(jnp.concatenate([dst_c, dst_w, dst_r + N]), E3P, 0)

    lgt, maxes = _build_sddmm()(qcat, kcat, srck, dstq)
    m_p = jnp.max(maxes[:NW])
    m_a = jnp.max(maxes[NW:])

    vcat_p = jnp.concatenate([v_c, v_w])
    vcat4_p = vcat_p.reshape(2 * N, NPASS, DP).transpose(1, 0, 2).reshape(-1, DP)
    srcv_p = _pad1(jnp.concatenate([src_c, src_w + N]), EPP, 0)
    dstv_p = _pad1(jnp.concatenate([dst_c, dst_w]), EPP, 0)
    lgt_p = _pad1(lgt[:2 * E], EPP, NEG)
    agg_p2, den_p2 = _make_agg(TPT_P, 2 * N)(vcat4_p, srcv_p, dstv_p, lgt_p,
                                             jnp.full((L,), m_p, jnp.float32))

    vcat4_a = v_r.reshape(N, NPASS, DP).transpose(1, 0, 2).reshape(-1, DP)
    srcv_a = _pad1(src_r, EAP, 0)
    dstv_a = _pad1(dst_r, EAP, 0)
    lgt_a = _pad1(lgt[2 * E:3 * E], EAP, NEG)
    agg_a2, den_a2 = _make_agg(TPT_A, N)(vcat4_a, srcv_a, dstv_a, lgt_a,
                                         jnp.full((L,), m_a, jnp.float32))

    out = {}
    for t, agg2, den2 in (('paper', agg_p2, den_p2), ('author', agg_a2, den_a2)):
        agg = (agg2.reshape(NPASS, NC, NPS, DP).sum(axis=1)[:, :N]
               .transpose(1, 0, 2).reshape(N, D))
        den = den2.reshape(NC, NPS).sum(axis=0)[:N]
        agg = agg / (den + 1e-16)[:, None]
        o = _lin(jax.nn.gelu(agg), lp['out'][t])
        sg = jax.nn.sigmoid(lp['skip'][t])
        out[t] = sg * o + (1.0 - sg) * xd[t]
    return out


def kernel(x_paper, x_author, edge_index_cites, edge_index_writes,
           edge_index_rev_writes, x_covs, batch_num, is_sparsed,
           class_batch_info, params):
    edges = {'cites': edge_index_cites, 'writes': edge_index_writes,
             'rev_writes': edge_index_rev_writes}
    xd = {
        'paper': _lin(x_paper, params['basis']['paper']),
        'author': _lin(x_author, params['basis']['author']),
    }
    for lp in params['layers']:
        xd = _hgt_layer_sc(xd, edges, lp)
        xd = {t: jax.nn.relu(v) for t, v in xd.items()}

    x = xd['paper']
    xpad = jnp.concatenate([x, jnp.zeros((NPOOL - N, D), jnp.float32)])
    cpad = _pad1(class_batch_info, NPOOL, 0)
    parts = _build_pool()(xpad, cpad)
    pooled = parts.reshape(NW, B, D).sum(axis=0)
    bounds = jnp.searchsorted(class_batch_info, jnp.arange(B + 1, dtype=jnp.int32))
    cnt = (bounds[1:] - bounds[:-1]).astype(jnp.float32)
    pooled = pooled / jnp.maximum(cnt, 1.0)[:, None]

    h = jnp.concatenate([pooled, x_covs], axis=1)
    h = jax.nn.relu(_lin(h, params['lin1']))
    return _lin(h, params['lin2'])


# speedup vs baseline: 3.9921x; 1.1707x over previous
import functools

import jax
import jax.numpy as jnp
from jax import lax
from jax.experimental import pallas as pl
from jax.experimental.pallas import tpu as pltpu
from jax.experimental.pallas import tpu_sc as plsc

N = 50000
D = 128
E = 200000
B = 64
L = 16
NC = 2
NS = 16
NW = NC * NS

NEG = -1e30

E3 = 3 * E
TPB1 = 18944
E3P = TPB1 * NW
NB1 = TPB1 // 128

CHUNK = 12800
NCH = 4
NPAD = CHUNK * NCH

RPT = 1568
NPOOL = RPT * NW
PBLK = 32
NPB = RPT // PBLK

def _pad1(x, n, val):
    return jnp.concatenate([x, jnp.full((n - x.shape[0],), val, x.dtype)])


def _lane_sum(x):
    iota = lax.iota(jnp.int32, L)
    for k in (1, 2, 4, 8):
        x = x + x.at[iota ^ k].get(mode="promise_in_bounds")
    return x


@functools.cache
def _build_sddmm():
  _mesh = plsc.VectorSubcoreMesh(core_axis_name="c", subcore_axis_name="s")

  @functools.partial(
    pl.kernel,
    out_type=(
        jax.ShapeDtypeStruct((E3P,), jnp.float32),
        jax.ShapeDtypeStruct((2 * NW, L), jnp.float32),
    ),
    mesh=_mesh,
    scratch_types=[
        pltpu.VMEM((128,), jnp.int32),
        pltpu.VMEM((128,), jnp.int32),
        pltpu.VMEM((128,), jnp.int32),
        pltpu.VMEM((128,), jnp.int32),
        pltpu.VMEM((128, D), jnp.float32),
        pltpu.VMEM((128, D), jnp.float32),
        pltpu.VMEM((128, D), jnp.float32),
        pltpu.VMEM((128, D), jnp.float32),
        pltpu.VMEM((128,), jnp.float32),
        pltpu.VMEM((128,), jnp.float32),
        pltpu.VMEM((L,), jnp.float32),
        pltpu.VMEM((L,), jnp.float32),
        pltpu.SemaphoreType.DMA,
        pltpu.SemaphoreType.DMA,
        pltpu.SemaphoreType.DMA,
    ],
  )
  def _sddmm(qcat, kcat, srck, dstq, lgt_out, max_out,
             srcA, dstA, srcB, dstB, krowsA, qrowsA, krowsB, qrowsB,
             lgtA, lgtB, mb0, mb1, semA, semB, semI):
    c = lax.axis_index("c")
    sx = lax.axis_index("s")
    wid = sx * NC + c
    base = wid * TPB1
    iota = lax.iota(jnp.int32, L)
    negv = jnp.full((L,), NEG, jnp.float32)

    def gather_issue(srcb, dstb, krows, qrows, sem):
        pltpu.async_copy(kcat.at[srcb], krows, sem)
        pltpu.async_copy(qcat.at[dstb], qrows, sem)

    def gather_wait(srcb, dstb, krows, qrows, sem):
        pltpu.make_async_copy(kcat.at[srcb], krows, sem).wait()
        pltpu.make_async_copy(qcat.at[dstb], qrows, sem).wait()

    def idx_issue(off, srcb, dstb):
        pltpu.async_copy(srck.at[pl.ds(off, 128)], srcb, semI)
        pltpu.async_copy(dstq.at[pl.ds(off, 128)], dstb, semI)

    def idx_wait(off, srcb, dstb):
        pltpu.make_async_copy(srck.at[pl.ds(off, 128)], srcb, semI).wait()
        pltpu.make_async_copy(dstq.at[pl.ds(off, 128)], dstb, semI).wait()

    def compute(off, dstb, krows, qrows, lgtb, maxp, maxa):
        for g in range(8):
            def d_body(j, accs):
                dj = j * 16
                return tuple(
                    accs[e] + qrows[g * 16 + e, pl.ds(dj, 16)]
                    * krows[g * 16 + e, pl.ds(dj, 16)]
                    for e in range(16)
                )

            zero = jnp.zeros((L,), jnp.float32)
            accs = lax.fori_loop(0, 8, d_body, (zero,) * 16)
            lgt = zero
            for e in range(16):
                lgt = jnp.where(iota == e, _lane_sum(accs[e]), lgt)
            ids = off + g * 16 + iota
            lgt = jnp.where(ids < E3, lgt, negv)
            dst16 = dstb[pl.ds(g * 16, 16)]
            isp = dst16 < N
            maxp = jnp.maximum(maxp, jnp.where(isp, lgt, negv))
            maxa = jnp.maximum(maxa, jnp.where(isp, negv, lgt))
            lgtb[pl.ds(g * 16, 16)] = lgt
        return maxp, maxa

    pltpu.sync_copy(srck.at[pl.ds(base, 128)], srcA)
    pltpu.sync_copy(dstq.at[pl.ds(base, 128)], dstA)
    gather_issue(srcA, dstA, krowsA, qrowsA, semA)

    def body(i, carry):
        maxp, maxa = carry
        b0 = 2 * i
        off0 = base + b0 * 128
        off1 = off0 + 128
        off2 = off0 + 256
        idx_issue(off1, srcB, dstB)
        gather_wait(srcA, dstA, krowsA, qrowsA, semA)
        idx_wait(off1, srcB, dstB)
        gather_issue(srcB, dstB, krowsB, qrowsB, semB)
        maxp, maxa = compute(off0, dstA, krowsA, qrowsA, lgtA, maxp, maxa)
        pltpu.sync_copy(lgtA, lgt_out.at[pl.ds(off0, 128)])

        @pl.when(b0 + 2 < NB1)
        def _():
            idx_issue(off2, srcA, dstA)
            idx_wait(off2, srcA, dstA)
            gather_issue(srcA, dstA, krowsA, qrowsA, semA)

        gather_wait(srcB, dstB, krowsB, qrowsB, semB)
        maxp, maxa = compute(off1, dstB, krowsB, qrowsB, lgtB, maxp, maxa)
        pltpu.sync_copy(lgtB, lgt_out.at[pl.ds(off1, 128)])
        return maxp, maxa

    mp, ma = lax.fori_loop(0, NB1 // 2, body, (negv, negv))
    mb0[...] = mp
    mb1[...] = ma
    pltpu.sync_copy(mb0, max_out.at[wid])
    pltpu.sync_copy(mb1, max_out.at[NW + wid])

  return _sddmm


NPS = 50176
RPS = NPS // NS
NPASS = 8
DP = D // NPASS


@functools.cache
def _make_agg(TPT, VT):
    _mesh = plsc.VectorSubcoreMesh(core_axis_name="c", subcore_axis_name="s")
    NBLK = TPT // 128

    @functools.partial(
        pl.kernel,
        out_type=(
            jax.ShapeDtypeStruct((NPASS * NC * NPS, DP), jnp.float32),
            jax.ShapeDtypeStruct((NC * NPS,), jnp.float32),
        ),
        mesh=_mesh,
        compiler_params=pltpu.CompilerParams(use_tc_tiling_on_sc=False),
        scratch_types=[
            pltpu.VMEM((TPT,), jnp.int32),
            pltpu.VMEM((TPT,), jnp.int32),
            pltpu.VMEM((TPT,), jnp.float32),
            pltpu.VMEM((128,), jnp.int32),
            pltpu.VMEM((128,), jnp.int32),
            pltpu.VMEM((128,), jnp.float32),
            pltpu.VMEM((128, DP), jnp.float32),
            pltpu.VMEM((128,), jnp.int32),
            pltpu.VMEM((128,), jnp.int32),
            pltpu.VMEM((128,), jnp.float32),
            pltpu.VMEM((128, DP), jnp.float32),
            pltpu.VMEM((L,), jnp.float32),
            pltpu.VMEM((392, DP), jnp.float32),
            pltpu.VMEM((392, DP), jnp.float32),
            pltpu.VMEM((RPS,), jnp.float32),
            pltpu.VMEM((RPS,), jnp.float32),
            pltpu.VMEM_SHARED((NPS, DP), jnp.float32),
            pltpu.VMEM_SHARED((NPS,), jnp.float32),
            pltpu.SemaphoreType.DMA,
            pltpu.SemaphoreType.DMA,
        ],
    )
    def _agg(vcat4, srcv, dstv, lgtv, mvec, agg_out, den_out,
             src_sh, dst_sh, lgt_sh, srcblkA, idxblkA, exblkA, rowsA,
             srcblkB, idxblkB, exblkB, rowsB, mv,
             zrows, stg, zden, stgd, sp_agg, sp_den, semA, semB):
        c = lax.axis_index("c")
        s = lax.axis_index("s")
        wid = s * NC + c
        base = wid * TPT
        z16 = jnp.zeros((L,), jnp.float32)

        pltpu.sync_copy(srcv.at[pl.ds(base, TPT)], src_sh)
        pltpu.sync_copy(dstv.at[pl.ds(base, TPT)], dst_sh)
        pltpu.sync_copy(lgtv.at[pl.ds(base, TPT)], lgt_sh)
        pltpu.sync_copy(mvec, mv)
        M = mv[...]

        def exf(i, _):
            sl = pl.ds(i * 16, 16)
            lgt_sh[sl] = jnp.exp(lgt_sh[sl] - M)
            return 0

        lax.fori_loop(0, TPT // 16, exf, 0)

        def zfill_rows(i, _):
            for j in range(DP // 16):
                zrows[i, pl.ds(j * 16, 16)] = z16
            return 0

        lax.fori_loop(0, 392, zfill_rows, 0)

        def zfill_den(i, _):
            zden[pl.ds(i * 16, 16)] = z16
            return 0

        lax.fori_loop(0, RPS // 16, zfill_den, 0)

        def build(boff, poff, srcblk, idxblk, exblk):
            for g in range(8):
                sl = pl.ds(boff + g * 16, 16)
                gsl = pl.ds(g * 16, 16)
                idxblk[gsl] = dst_sh[sl]
                srcblk[gsl] = src_sh[sl] + poff
                exblk[gsl] = lgt_sh[sl]

        def scale_store(p, srcblk, idxblk, exblk, rows, sem):
            pltpu.make_async_copy(vcat4.at[srcblk], rows, sem).wait()
            for g in range(8):
                ex = exblk[pl.ds(g * 16, 16)]
                for e in range(16):
                    r = g * 16 + e
                    exs = ex[e]
                    for j in range(DP // 16):
                        jsl = pl.ds(j * 16, 16)
                        rows[r, jsl] = rows[r, jsl] * exs
            pltpu.sync_copy(rows, sp_agg.at[idxblk], add=True)

            @pl.when(p == 0)
            def _():
                pltpu.sync_copy(exblk, sp_den.at[idxblk], add=True)

        def pass_body(p, _p):
            poff = p * VT
            for r in range(8):
                pltpu.sync_copy(zrows, sp_agg.at[pl.ds(s * RPS + r * 392, 392)])

            @pl.when(p == 0)
            def _():
                pltpu.sync_copy(zden, sp_den.at[pl.ds(s * RPS, RPS)])

            plsc.subcore_barrier()

            build(0, poff, srcblkA, idxblkA, exblkA)
            pltpu.async_copy(vcat4.at[srcblkA], rowsA, semA)

            def blk(i, _b):
                boff0 = 2 * i * 128
                build(boff0 + 128, poff, srcblkB, idxblkB, exblkB)
                pltpu.async_copy(vcat4.at[srcblkB], rowsB, semB)
                scale_store(p, srcblkA, idxblkA, exblkA, rowsA, semA)

                @pl.when(boff0 + 256 < TPT)
                def _():
                    build(boff0 + 256, poff, srcblkA, idxblkA, exblkA)
                    pltpu.async_copy(vcat4.at[srcblkA], rowsA, semA)

                scale_store(p, srcblkB, idxblkB, exblkB, rowsB, semB)
                return 0

            lax.fori_loop(0, NBLK // 2, blk, 0)
            plsc.subcore_barrier()
            obase = p * (NC * NPS) + c * NPS + s * RPS
            for r in range(8):
                pltpu.sync_copy(sp_agg.at[pl.ds(s * RPS + r * 392, 392)], stg)
                pltpu.sync_copy(stg, agg_out.at[pl.ds(obase + r * 392, 392)])

            @pl.when(p == 0)
            def _():
                pltpu.sync_copy(sp_den.at[pl.ds(s * RPS, RPS)], stgd)
                pltpu.sync_copy(stgd, den_out.at[pl.ds(c * NPS + s * RPS, RPS)])

            plsc.subcore_barrier()
            return 0

        lax.fori_loop(0, NPASS, pass_body, 0)

    return _agg


TPT_P = 12544
TPT_A = 6400
EPP = TPT_P * NW
EAP = TPT_A * NW


@functools.cache
def _build_pool():
  _mesh = plsc.VectorSubcoreMesh(core_axis_name="c", subcore_axis_name="s")

  @functools.partial(
    pl.kernel,
    out_type=jax.ShapeDtypeStruct((NW * B, D), jnp.float32),
    mesh=_mesh,
    scratch_types=[
        pltpu.VMEM((B, D), jnp.float32),
        pltpu.VMEM((PBLK, D), jnp.float32),
        pltpu.VMEM((PBLK + 16,), jnp.int32),
    ],
  )
  def _pool(x, cbi, out, acc, rbuf, cbuf):
    c = lax.axis_index("c")
    s = lax.axis_index("s")
    wid = s * NC + c
    base = wid * RPT
    z16 = jnp.zeros((L,), jnp.float32)

    def zf(i, _):
        for j in range(8):
            acc[i, pl.ds(j * 16, 16)] = z16
        return 0

    lax.fori_loop(0, B, zf, 0)

    def blk(b, _):
        off = base + b * PBLK
        pltpu.sync_copy(x.at[pl.ds(off, PBLK)], rbuf)
        pltpu.sync_copy(cbi.at[pl.ds(off, PBLK)], cbuf.at[pl.ds(0, PBLK)])

        def row(r, _2):
            cb = cbuf[pl.ds(r, 16)][0]
            for j in range(8):
                acc[cb, pl.ds(j * 16, 16)] = (acc[cb, pl.ds(j * 16, 16)]
                                              + rbuf[r, pl.ds(j * 16, 16)])
            return 0

        lax.fori_loop(0, PBLK, row, 0)
        return 0

    lax.fori_loop(0, NPB, blk, 0)
    pltpu.sync_copy(acc, out.at[pl.ds(wid * B, B)])

  return _pool


def _lin(x, wb):
    return x @ wb[0] + wb[1]


def _hgt_layer_sc(xd, edges, lp):
    sq = jnp.sqrt(jnp.float32(D))
    kk = {t: _lin(xd[t], lp['k'][t]) for t in ('paper', 'author')}
    qq = {t: _lin(xd[t], lp['q'][t]) for t in ('paper', 'author')}
    vv = {t: _lin(xd[t], lp['v'][t]) for t in ('paper', 'author')}

    k_c = (kk['paper'] @ lp['a_rel']['cites']) * (lp['p_rel']['cites'] / sq)
    k_w = (kk['author'] @ lp['a_rel']['writes']) * (lp['p_rel']['writes'] / sq)
    k_r = (kk['paper'] @ lp['a_rel']['rev_writes']) * (lp['p_rel']['rev_writes'] / sq)
    v_c = vv['paper'] @ lp['m_rel']['cites']
    v_w = vv['author'] @ lp['m_rel']['writes']
    v_r = vv['paper'] @ lp['m_rel']['rev_writes']

    qcat = jnp.concatenate([qq['paper'], qq['author']])
    kcat = jnp.concatenate([k_c, k_w, k_r])

    src_c, dst_c = edges['cites'][0], edges['cites'][1]
    src_w, dst_w = edges['writes'][0], edges['writes'][1]
    src_r, dst_r = edges['rev_writes'][0], edges['rev_writes'][1]

    srck = _pad1(jnp.concatenate([src_c, src_w + N, src_r + 2 * N]), E3P, 0)
    dstq = _pad1(jnp.concatenate([dst_c, dst_w, dst_r + N]), E3P, 0)

    lgt, maxes = _build_sddmm()(qcat, kcat, srck, dstq)
    m_p = jnp.max(maxes[:NW])
    m_a = jnp.max(maxes[NW:])

    vcat_p = jnp.concatenate([v_c, v_w])
    vcat4_p = vcat_p.reshape(2 * N, NPASS, DP).transpose(1, 0, 2).reshape(-1, DP)
    srcv_p = _pad1(jnp.concatenate([src_c, src_w + N]), EPP, 0)
    dstv_p = _pad1(jnp.concatenate([dst_c, dst_w]), EPP, 0)
    lgt_p = _pad1(lgt[:2 * E], EPP, NEG)
    agg_p2, den_p2 = _make_agg(TPT_P, 2 * N)(vcat4_p, srcv_p, dstv_p, lgt_p,
                                             jnp.full((L,), m_p, jnp.float32))

    vcat4_a = v_r.reshape(N, NPASS, DP).transpose(1, 0, 2).reshape(-1, DP)
    srcv_a = _pad1(src_r, EAP, 0)
    dstv_a = _pad1(dst_r, EAP, 0)
    lgt_a = _pad1(lgt[2 * E:3 * E], EAP, NEG)
    agg_a2, den_a2 = _make_agg(TPT_A, N)(vcat4_a, srcv_a, dstv_a, lgt_a,
                                         jnp.full((L,), m_a, jnp.float32))

    out = {}
    for t, agg2, den2 in (('paper', agg_p2, den_p2), ('author', agg_a2, den_a2)):
        agg = (agg2.reshape(NPASS, NC, NPS, DP).sum(axis=1)[:, :N]
               .transpose(1, 0, 2).reshape(N, D))
        den = den2.reshape(NC, NPS).sum(axis=0)[:N]
        agg = agg / (den + 1e-16)[:, None]
        o = _lin(jax.nn.gelu(agg), lp['out'][t])
        sg = jax.nn.sigmoid(lp['skip'][t])
        out[t] = sg * o + (1.0 - sg) * xd[t]
    return out


def kernel(x_paper, x_author, edge_index_cites, edge_index_writes,
           edge_index_rev_writes, x_covs, batch_num, is_sparsed,
           class_batch_info, params):
    edges = {'cites': edge_index_cites, 'writes': edge_index_writes,
             'rev_writes': edge_index_rev_writes}
    xd = {
        'paper': _lin(x_paper, params['basis']['paper']),
        'author': _lin(x_author, params['basis']['author']),
    }
    for lp in params['layers']:
        xd = _hgt_layer_sc(xd, edges, lp)
        xd = {t: jax.nn.relu(v) for t, v in xd.items()}

    x = xd['paper']
    xpad = jnp.concatenate([x, jnp.zeros((NPOOL - N, D), jnp.float32)])
    cpad = _pad1(class_batch_info, NPOOL, 0)
    parts = _build_pool()(xpad, cpad)
    pooled = parts.reshape(NW, B, D).sum(axis=0)
    bounds = jnp.searchsorted(class_batch_info, jnp.arange(B + 1, dtype=jnp.int32))
    cnt = (bounds[1:] - bounds[:-1]).astype(jnp.float32)
    pooled = pooled / jnp.maximum(cnt, 1.0)[:, None]

    h = jnp.concatenate([pooled, x_covs], axis=1)
    h = jax.nn.relu(_lin(h, params['lin1']))
    return _lin(h, params['lin2'])


# speedup vs baseline: 4.0734x; 1.0204x over previous
import functools

import jax
import jax.numpy as jnp
from jax import lax
from jax.experimental import pallas as pl
from jax.experimental.pallas import tpu as pltpu
from jax.experimental.pallas import tpu_sc as plsc

N = 50000
D = 128
E = 200000
B = 64
L = 16
NC = 2
NS = 16
NW = NC * NS

NEG = -1e30

E3 = 3 * E
TPB1 = 18944
E3P = TPB1 * NW
NB1 = TPB1 // 128

CHUNK = 12800
NCH = 4
NPAD = CHUNK * NCH

RPT = 1568
NPOOL = RPT * NW
PBLK = 32
NPB = RPT // PBLK

def _pad1(x, n, val):
    return jnp.concatenate([x, jnp.full((n - x.shape[0],), val, x.dtype)])


def _lane_sum(x):
    iota = lax.iota(jnp.int32, L)
    for k in (1, 2, 4, 8):
        x = x + x.at[iota ^ k].get(mode="promise_in_bounds")
    return x


@functools.cache
def _build_sddmm():
  _mesh = plsc.VectorSubcoreMesh(core_axis_name="c", subcore_axis_name="s")

  @functools.partial(
    pl.kernel,
    out_type=(
        jax.ShapeDtypeStruct((E3P,), jnp.float32),
        jax.ShapeDtypeStruct((2 * NW, L), jnp.float32),
    ),
    mesh=_mesh,
    scratch_types=[
        pltpu.VMEM((128,), jnp.int32),
        pltpu.VMEM((128,), jnp.int32),
        pltpu.VMEM((128,), jnp.int32),
        pltpu.VMEM((128,), jnp.int32),
        pltpu.VMEM((128, D), jnp.float32),
        pltpu.VMEM((128, D), jnp.float32),
        pltpu.VMEM((128, D), jnp.float32),
        pltpu.VMEM((128, D), jnp.float32),
        pltpu.VMEM((128,), jnp.float32),
        pltpu.VMEM((128,), jnp.float32),
        pltpu.VMEM((L,), jnp.float32),
        pltpu.VMEM((L,), jnp.float32),
        pltpu.SemaphoreType.DMA,
        pltpu.SemaphoreType.DMA,
        pltpu.SemaphoreType.DMA,
    ],
  )
  def _sddmm(qcat, kcat, srck, dstq, lgt_out, max_out,
             srcA, dstA, srcB, dstB, krowsA, qrowsA, krowsB, qrowsB,
             lgtA, lgtB, mb0, mb1, semA, semB, semI):
    c = lax.axis_index("c")
    sx = lax.axis_index("s")
    wid = sx * NC + c
    base = wid * TPB1
    iota = lax.iota(jnp.int32, L)
    negv = jnp.full((L,), NEG, jnp.float32)

    def gather_issue(srcb, dstb, krows, qrows, sem):
        pltpu.async_copy(kcat.at[srcb], krows, sem)
        pltpu.async_copy(qcat.at[dstb], qrows, sem)

    def gather_wait(srcb, dstb, krows, qrows, sem):
        pltpu.make_async_copy(kcat.at[srcb], krows, sem).wait()
        pltpu.make_async_copy(qcat.at[dstb], qrows, sem).wait()

    def idx_issue(off, srcb, dstb):
        pltpu.async_copy(srck.at[pl.ds(off, 128)], srcb, semI)
        pltpu.async_copy(dstq.at[pl.ds(off, 128)], dstb, semI)

    def idx_wait(off, srcb, dstb):
        pltpu.make_async_copy(srck.at[pl.ds(off, 128)], srcb, semI).wait()
        pltpu.make_async_copy(dstq.at[pl.ds(off, 128)], dstb, semI).wait()

    def compute(off, dstb, krows, qrows, lgtb, maxp, maxa):
        for g in range(8):
            def d_body(j, accs):
                dj = j * 16
                return tuple(
                    accs[e] + qrows[g * 16 + e, pl.ds(dj, 16)]
                    * krows[g * 16 + e, pl.ds(dj, 16)]
                    for e in range(16)
                )

            zero = jnp.zeros((L,), jnp.float32)
            accs = lax.fori_loop(0, 8, d_body, (zero,) * 16)
            lgt = zero
            for e in range(16):
                lgt = jnp.where(iota == e, _lane_sum(accs[e]), lgt)
            ids = off + g * 16 + iota
            lgt = jnp.where(ids < E3, lgt, negv)
            dst16 = dstb[pl.ds(g * 16, 16)]
            isp = dst16 < N
            maxp = jnp.maximum(maxp, jnp.where(isp, lgt, negv))
            maxa = jnp.maximum(maxa, jnp.where(isp, negv, lgt))
            lgtb[pl.ds(g * 16, 16)] = lgt
        return maxp, maxa

    pltpu.sync_copy(srck.at[pl.ds(base, 128)], srcA)
    pltpu.sync_copy(dstq.at[pl.ds(base, 128)], dstA)
    gather_issue(srcA, dstA, krowsA, qrowsA, semA)

    def body(i, carry):
        maxp, maxa = carry
        b0 = 2 * i
        off0 = base + b0 * 128
        off1 = off0 + 128
        off2 = off0 + 256
        idx_issue(off1, srcB, dstB)
        gather_wait(srcA, dstA, krowsA, qrowsA, semA)
        idx_wait(off1, srcB, dstB)
        gather_issue(srcB, dstB, krowsB, qrowsB, semB)
        maxp, maxa = compute(off0, dstA, krowsA, qrowsA, lgtA, maxp, maxa)
        pltpu.sync_copy(lgtA, lgt_out.at[pl.ds(off0, 128)])

        @pl.when(b0 + 2 < NB1)
        def _():
            idx_issue(off2, srcA, dstA)
            idx_wait(off2, srcA, dstA)
            gather_issue(srcA, dstA, krowsA, qrowsA, semA)

        gather_wait(srcB, dstB, krowsB, qrowsB, semB)
        maxp, maxa = compute(off1, dstB, krowsB, qrowsB, lgtB, maxp, maxa)
        pltpu.sync_copy(lgtB, lgt_out.at[pl.ds(off1, 128)])
        return maxp, maxa

    mp, ma = lax.fori_loop(0, NB1 // 2, body, (negv, negv))
    mb0[...] = mp
    mb1[...] = ma
    pltpu.sync_copy(mb0, max_out.at[wid])
    pltpu.sync_copy(mb1, max_out.at[NW + wid])

  return _sddmm


NPS = 50176
RPS = NPS // NS
NPASS = 8
DP = D // NPASS


@functools.cache
def _make_agg(TPT, VT):
    _mesh = plsc.VectorSubcoreMesh(core_axis_name="c", subcore_axis_name="s")
    NBLK = TPT // 128

    @functools.partial(
        pl.kernel,
        out_type=(
            jax.ShapeDtypeStruct((NPASS * NC * NPS, DP), jnp.float32),
            jax.ShapeDtypeStruct((NC * NPS,), jnp.float32),
        ),
        mesh=_mesh,
        compiler_params=pltpu.CompilerParams(use_tc_tiling_on_sc=False),
        scratch_types=[
            pltpu.VMEM((TPT,), jnp.int32),
            pltpu.VMEM((TPT,), jnp.int32),
            pltpu.VMEM((TPT,), jnp.float32),
            pltpu.VMEM((128,), jnp.int32),
            pltpu.VMEM((128,), jnp.int32),
            pltpu.VMEM((128,), jnp.float32),
            pltpu.VMEM((128, DP), jnp.float32),
            pltpu.VMEM((128,), jnp.int32),
            pltpu.VMEM((128,), jnp.int32),
            pltpu.VMEM((128,), jnp.float32),
            pltpu.VMEM((128, DP), jnp.float32),
            pltpu.VMEM((L,), jnp.float32),
            pltpu.VMEM((392, DP), jnp.float32),
            pltpu.VMEM((392, DP), jnp.float32),
            pltpu.VMEM((RPS,), jnp.float32),
            pltpu.VMEM((RPS,), jnp.float32),
            pltpu.VMEM_SHARED((NPS, DP), jnp.float32),
            pltpu.VMEM_SHARED((NPS,), jnp.float32),
            pltpu.SemaphoreType.DMA,
            pltpu.SemaphoreType.DMA,
        ],
    )
    def _agg(vcat4, srcv, dstv, lgtv, mvec, agg_out, den_out,
             src_sh, dst_sh, lgt_sh, srcblkA, idxblkA, exblkA, rowsA,
             srcblkB, idxblkB, exblkB, rowsB, mv,
             zrows, stg, zden, stgd, sp_agg, sp_den, semA, semB):
        c = lax.axis_index("c")
        s = lax.axis_index("s")
        wid = s * NC + c
        base = wid * TPT
        z16 = jnp.zeros((L,), jnp.float32)

        pltpu.sync_copy(srcv.at[pl.ds(base, TPT)], src_sh)
        pltpu.sync_copy(dstv.at[pl.ds(base, TPT)], dst_sh)
        pltpu.sync_copy(lgtv.at[pl.ds(base, TPT)], lgt_sh)
        pltpu.sync_copy(mvec, mv)
        M = mv[...]

        def exf(i, _):
            sl = pl.ds(i * 16, 16)
            lgt_sh[sl] = jnp.exp(lgt_sh[sl] - M)
            return 0

        lax.fori_loop(0, TPT // 16, exf, 0)

        def zfill_rows(i, _):
            for j in range(DP // 16):
                zrows[i, pl.ds(j * 16, 16)] = z16
            return 0

        lax.fori_loop(0, 392, zfill_rows, 0)

        def zfill_den(i, _):
            zden[pl.ds(i * 16, 16)] = z16
            return 0

        lax.fori_loop(0, RPS // 16, zfill_den, 0)

        def build(boff, poff, srcblk, idxblk, exblk):
            for g in range(8):
                sl = pl.ds(boff + g * 16, 16)
                gsl = pl.ds(g * 16, 16)
                idxblk[gsl] = dst_sh[sl]
                srcblk[gsl] = src_sh[sl] + poff
                exblk[gsl] = lgt_sh[sl]

        def scale_store(p, srcblk, idxblk, exblk, rows, sem):
            pltpu.make_async_copy(vcat4.at[srcblk], rows, sem).wait()
            for g in range(8):
                ex = exblk[pl.ds(g * 16, 16)]
                for e in range(16):
                    r = g * 16 + e
                    exs = ex[e]
                    for j in range(DP // 16):
                        jsl = pl.ds(j * 16, 16)
                        rows[r, jsl] = rows[r, jsl] * exs
            pltpu.sync_copy(rows, sp_agg.at[idxblk], add=True)

            @pl.when(p == 0)
            def _():
                pltpu.sync_copy(exblk, sp_den.at[idxblk], add=True)

        def pass_body(p, _p):
            poff = p * VT
            for r in range(8):
                pltpu.sync_copy(zrows, sp_agg.at[pl.ds(s * RPS + r * 392, 392)])

            @pl.when(p == 0)
            def _():
                pltpu.sync_copy(zden, sp_den.at[pl.ds(s * RPS, RPS)])

            plsc.subcore_barrier()

            build(0, poff, srcblkA, idxblkA, exblkA)
            pltpu.async_copy(vcat4.at[srcblkA], rowsA, semA)

            def blk(i, _b):
                boff0 = 2 * i * 128
                build(boff0 + 128, poff, srcblkB, idxblkB, exblkB)
                pltpu.async_copy(vcat4.at[srcblkB], rowsB, semB)
                scale_store(p, srcblkA, idxblkA, exblkA, rowsA, semA)

                @pl.when(boff0 + 256 < TPT)
                def _():
                    build(boff0 + 256, poff, srcblkA, idxblkA, exblkA)
                    pltpu.async_copy(vcat4.at[srcblkA], rowsA, semA)

                scale_store(p, srcblkB, idxblkB, exblkB, rowsB, semB)
                return 0

            lax.fori_loop(0, NBLK // 2, blk, 0)
            plsc.subcore_barrier()
            obase = p * (NC * NPS) + c * NPS + s * RPS
            for r in range(8):
                pltpu.sync_copy(sp_agg.at[pl.ds(s * RPS + r * 392, 392)], stg)
                pltpu.sync_copy(stg, agg_out.at[pl.ds(obase + r * 392, 392)])

            @pl.when(p == 0)
            def _():
                pltpu.sync_copy(sp_den.at[pl.ds(s * RPS, RPS)], stgd)
                pltpu.sync_copy(stgd, den_out.at[pl.ds(c * NPS + s * RPS, RPS)])

            plsc.subcore_barrier()
            return 0

        lax.fori_loop(0, NPASS, pass_body, 0)

    return _agg


TPT_P = 12544
TPT_A = 6400
EPP = TPT_P * NW
EAP = TPT_A * NW


@functools.cache
def _build_pool():
  _mesh = plsc.VectorSubcoreMesh(core_axis_name="c", subcore_axis_name="s")

  @functools.partial(
    pl.kernel,
    out_type=jax.ShapeDtypeStruct((NC * B, D), jnp.float32),
    mesh=_mesh,
    scratch_types=[
        pltpu.VMEM((PBLK, D), jnp.float32),
        pltpu.VMEM((PBLK, D), jnp.float32),
        pltpu.VMEM((PBLK,), jnp.int32),
        pltpu.VMEM((PBLK,), jnp.int32),
        pltpu.VMEM((B, D), jnp.float32),
        pltpu.VMEM_SHARED((B, D), jnp.float32),
        pltpu.SemaphoreType.DMA,
        pltpu.SemaphoreType.DMA,
    ],
  )
  def _pool(x, cbi, out, rbufA, rbufB, cbufA, cbufB, zb, sp_pool, semA, semB):
    c = lax.axis_index("c")
    s = lax.axis_index("s")
    wid = s * NC + c
    base = wid * RPT
    z16 = jnp.zeros((L,), jnp.float32)

    def zf(i, _):
        for j in range(D // 16):
            zb[i, pl.ds(j * 16, 16)] = z16
        return 0

    lax.fori_loop(0, B, zf, 0)

    @pl.when(s == 0)
    def _():
        pltpu.sync_copy(zb, sp_pool)

    plsc.subcore_barrier()

    def issue(off, rbuf, cbuf, sem):
        pltpu.async_copy(x.at[pl.ds(off, PBLK)], rbuf, sem)
        pltpu.async_copy(cbi.at[pl.ds(off, PBLK)], cbuf, sem)

    def drain_add(off, rbuf, cbuf, sem):
        pltpu.make_async_copy(x.at[pl.ds(off, PBLK)], rbuf, sem).wait()
        pltpu.make_async_copy(cbi.at[pl.ds(off, PBLK)], cbuf, sem).wait()
        pltpu.sync_copy(rbuf, sp_pool.at[cbuf], add=True)

    issue(base, rbufA, cbufA, semA)

    def blk(i, _):
        off0 = base + 2 * i * PBLK
        issue(off0 + PBLK, rbufB, cbufB, semB)
        drain_add(off0, rbufA, cbufA, semA)

        @pl.when(2 * i + 2 < NPB)
        def _():
            issue(off0 + 2 * PBLK, rbufA, cbufA, semA)

        drain_add(off0 + PBLK, rbufB, cbufB, semB)
        return 0

    lax.fori_loop(0, NPB // 2, blk, 0)
    off_t = base + (NPB - 1) * PBLK
    drain_add(off_t, rbufA, cbufA, semA)
    plsc.subcore_barrier()

    @pl.when(s == 0)
    def _():
        pltpu.sync_copy(sp_pool, zb)
        pltpu.sync_copy(zb, out.at[pl.ds(c * B, B)])

  return _pool


def _lin(x, wb):
    return x @ wb[0] + wb[1]


def _hgt_layer_sc(xd, edges, lp):
    sq = jnp.sqrt(jnp.float32(D))
    kk = {t: _lin(xd[t], lp['k'][t]) for t in ('paper', 'author')}
    qq = {t: _lin(xd[t], lp['q'][t]) for t in ('paper', 'author')}
    vv = {t: _lin(xd[t], lp['v'][t]) for t in ('paper', 'author')}

    k_c = (kk['paper'] @ lp['a_rel']['cites']) * (lp['p_rel']['cites'] / sq)
    k_w = (kk['author'] @ lp['a_rel']['writes']) * (lp['p_rel']['writes'] / sq)
    k_r = (kk['paper'] @ lp['a_rel']['rev_writes']) * (lp['p_rel']['rev_writes'] / sq)
    v_c = vv['paper'] @ lp['m_rel']['cites']
    v_w = vv['author'] @ lp['m_rel']['writes']
    v_r = vv['paper'] @ lp['m_rel']['rev_writes']

    qcat = jnp.concatenate([qq['paper'], qq['author']])
    kcat = jnp.concatenate([k_c, k_w, k_r])

    src_c, dst_c = edges['cites'][0], edges['cites'][1]
    src_w, dst_w = edges['writes'][0], edges['writes'][1]
    src_r, dst_r = edges['rev_writes'][0], edges['rev_writes'][1]

    srck = _pad1(jnp.concatenate([src_c, src_w + N, src_r + 2 * N]), E3P, 0)
    dstq = _pad1(jnp.concatenate([dst_c, dst_w, dst_r + N]), E3P, 0)

    lgt, maxes = _build_sddmm()(qcat, kcat, srck, dstq)
    m_p = jnp.max(maxes[:NW])
    m_a = jnp.max(maxes[NW:])

    vcat_p = jnp.concatenate([v_c, v_w])
    vcat4_p = vcat_p.reshape(2 * N, NPASS, DP).transpose(1, 0, 2).reshape(-1, DP)
    srcv_p = _pad1(jnp.concatenate([src_c, src_w + N]), EPP, 0)
    dstv_p = _pad1(jnp.concatenate([dst_c, dst_w]), EPP, 0)
    lgt_p = _pad1(lgt[:2 * E], EPP, NEG)
    agg_p2, den_p2 = _make_agg(TPT_P, 2 * N)(vcat4_p, srcv_p, dstv_p, lgt_p,
                                             jnp.full((L,), m_p, jnp.float32))

    vcat4_a = v_r.reshape(N, NPASS, DP).transpose(1, 0, 2).reshape(-1, DP)
    srcv_a = _pad1(src_r, EAP, 0)
    dstv_a = _pad1(dst_r, EAP, 0)
    lgt_a = _pad1(lgt[2 * E:3 * E], EAP, NEG)
    agg_a2, den_a2 = _make_agg(TPT_A, N)(vcat4_a, srcv_a, dstv_a, lgt_a,
                                         jnp.full((L,), m_a, jnp.float32))

    out = {}
    for t, agg2, den2 in (('paper', agg_p2, den_p2), ('author', agg_a2, den_a2)):
        agg = (agg2.reshape(NPASS, NC, NPS, DP).sum(axis=1)[:, :N]
               .transpose(1, 0, 2).reshape(N, D))
        den = den2.reshape(NC, NPS).sum(axis=0)[:N]
        agg = agg / (den + 1e-16)[:, None]
        o = _lin(jax.nn.gelu(agg), lp['out'][t])
        sg = jax.nn.sigmoid(lp['skip'][t])
        out[t] = sg * o + (1.0 - sg) * xd[t]
    return out


def kernel(x_paper, x_author, edge_index_cites, edge_index_writes,
           edge_index_rev_writes, x_covs, batch_num, is_sparsed,
           class_batch_info, params):
    edges = {'cites': edge_index_cites, 'writes': edge_index_writes,
             'rev_writes': edge_index_rev_writes}
    xd = {
        'paper': _lin(x_paper, params['basis']['paper']),
        'author': _lin(x_author, params['basis']['author']),
    }
    for lp in params['layers']:
        xd = _hgt_layer_sc(xd, edges, lp)
        xd = {t: jax.nn.relu(v) for t, v in xd.items()}

    x = xd['paper']
    xpad = jnp.concatenate([x, jnp.zeros((NPOOL - N, D), jnp.float32)])
    cpad = _pad1(class_batch_info, NPOOL, 0)
    parts = _build_pool()(xpad, cpad)
    pooled = parts.reshape(NC, B, D).sum(axis=0)
    bounds = jnp.searchsorted(class_batch_info, jnp.arange(B + 1, dtype=jnp.int32))
    cnt = (bounds[1:] - bounds[:-1]).astype(jnp.float32)
    pooled = pooled / jnp.maximum(cnt, 1.0)[:, None]

    h = jnp.concatenate([pooled, x_covs], axis=1)
    h = jax.nn.relu(_lin(h, params['lin1']))
    return _lin(h, params['lin2'])


# speedup vs baseline: 4.0777x; 1.0011x over previous
import functools

import jax
import jax.numpy as jnp
from jax import lax
from jax.experimental import pallas as pl
from jax.experimental.pallas import tpu as pltpu
from jax.experimental.pallas import tpu_sc as plsc

N = 50000
D = 128
E = 200000
B = 64
L = 16
NC = 2
NS = 16
NW = NC * NS

NEG = -1e30

E3 = 3 * E
TPB1 = 18944
E3P = TPB1 * NW
NB1 = TPB1 // 128

CHUNK = 12800
NCH = 4
NPAD = CHUNK * NCH

RPT = 1568
NPOOL = RPT * NW
PBLK = 32
NPB = RPT // PBLK

def _pad1(x, n, val):
    return jnp.concatenate([x, jnp.full((n - x.shape[0],), val, x.dtype)])


def _lane_sum(x):
    iota = lax.iota(jnp.int32, L)
    for k in (1, 2, 4, 8):
        x = x + x.at[iota ^ k].get(mode="promise_in_bounds")
    return x


@functools.cache
def _build_sddmm():
  _mesh = plsc.VectorSubcoreMesh(core_axis_name="c", subcore_axis_name="s")

  @functools.partial(
    pl.kernel,
    out_type=(
        jax.ShapeDtypeStruct((E3P,), jnp.float32),
        jax.ShapeDtypeStruct((2 * NW, L), jnp.float32),
    ),
    mesh=_mesh,
    scratch_types=[
        pltpu.VMEM((128,), jnp.int32),
        pltpu.VMEM((128,), jnp.int32),
        pltpu.VMEM((128,), jnp.int32),
        pltpu.VMEM((128,), jnp.int32),
        pltpu.VMEM((128, D), jnp.float32),
        pltpu.VMEM((128, D), jnp.float32),
        pltpu.VMEM((128, D), jnp.float32),
        pltpu.VMEM((128, D), jnp.float32),
        pltpu.VMEM((128,), jnp.float32),
        pltpu.VMEM((128,), jnp.float32),
        pltpu.VMEM((L,), jnp.float32),
        pltpu.VMEM((L,), jnp.float32),
        pltpu.SemaphoreType.DMA,
        pltpu.SemaphoreType.DMA,
        pltpu.SemaphoreType.DMA,
    ],
  )
  def _sddmm(qcat, kcat, srck, dstq, lgt_out, max_out,
             srcA, dstA, srcB, dstB, krowsA, qrowsA, krowsB, qrowsB,
             lgtA, lgtB, mb0, mb1, semA, semB, semI):
    c = lax.axis_index("c")
    sx = lax.axis_index("s")
    wid = sx * NC + c
    base = wid * TPB1
    iota = lax.iota(jnp.int32, L)
    negv = jnp.full((L,), NEG, jnp.float32)

    def gather_issue(srcb, dstb, krows, qrows, sem):
        pltpu.async_copy(kcat.at[srcb], krows, sem)
        pltpu.async_copy(qcat.at[dstb], qrows, sem)

    def gather_wait(srcb, dstb, krows, qrows, sem):
        pltpu.make_async_copy(kcat.at[srcb], krows, sem).wait()
        pltpu.make_async_copy(qcat.at[dstb], qrows, sem).wait()

    def idx_issue(off, srcb, dstb):
        pltpu.async_copy(srck.at[pl.ds(off, 128)], srcb, semI)
        pltpu.async_copy(dstq.at[pl.ds(off, 128)], dstb, semI)

    def idx_wait(off, srcb, dstb):
        pltpu.make_async_copy(srck.at[pl.ds(off, 128)], srcb, semI).wait()
        pltpu.make_async_copy(dstq.at[pl.ds(off, 128)], dstb, semI).wait()

    def compute(off, dstb, krows, qrows, lgtb, maxp, maxa):
        for g in range(8):
            def d_body(j, accs):
                dj = j * 16
                return tuple(
                    accs[e] + qrows[g * 16 + e, pl.ds(dj, 16)]
                    * krows[g * 16 + e, pl.ds(dj, 16)]
                    for e in range(16)
                )

            zero = jnp.zeros((L,), jnp.float32)
            accs = lax.fori_loop(0, 8, d_body, (zero,) * 16)
            vs = list(accs)
            k = 1
            while len(vs) > 1:
                mask = (iota & k) == 0
                nxt = []
                for j in range(0, len(vs), 2):
                    a, b = vs[j], vs[j + 1]
                    af = a + a.at[iota ^ k].get(mode="promise_in_bounds")
                    bf = b + b.at[iota ^ k].get(mode="promise_in_bounds")
                    nxt.append(jnp.where(mask, af, bf))
                vs = nxt
                k *= 2
            lgt = vs[0]
            ids = off + g * 16 + iota
            lgt = jnp.where(ids < E3, lgt, negv)
            dst16 = dstb[pl.ds(g * 16, 16)]
            isp = dst16 < N
            maxp = jnp.maximum(maxp, jnp.where(isp, lgt, negv))
            maxa = jnp.maximum(maxa, jnp.where(isp, negv, lgt))
            lgtb[pl.ds(g * 16, 16)] = lgt
        return maxp, maxa

    pltpu.sync_copy(srck.at[pl.ds(base, 128)], srcA)
    pltpu.sync_copy(dstq.at[pl.ds(base, 128)], dstA)
    gather_issue(srcA, dstA, krowsA, qrowsA, semA)

    def body(i, carry):
        maxp, maxa = carry
        b0 = 2 * i
        off0 = base + b0 * 128
        off1 = off0 + 128
        off2 = off0 + 256
        idx_issue(off1, srcB, dstB)
        gather_wait(srcA, dstA, krowsA, qrowsA, semA)
        idx_wait(off1, srcB, dstB)
        gather_issue(srcB, dstB, krowsB, qrowsB, semB)
        maxp, maxa = compute(off0, dstA, krowsA, qrowsA, lgtA, maxp, maxa)
        pltpu.sync_copy(lgtA, lgt_out.at[pl.ds(off0, 128)])

        @pl.when(b0 + 2 < NB1)
        def _():
            idx_issue(off2, srcA, dstA)
            idx_wait(off2, srcA, dstA)
            gather_issue(srcA, dstA, krowsA, qrowsA, semA)

        gather_wait(srcB, dstB, krowsB, qrowsB, semB)
        maxp, maxa = compute(off1, dstB, krowsB, qrowsB, lgtB, maxp, maxa)
        pltpu.sync_copy(lgtB, lgt_out.at[pl.ds(off1, 128)])
        return maxp, maxa

    mp, ma = lax.fori_loop(0, NB1 // 2, body, (negv, negv))
    mb0[...] = mp
    mb1[...] = ma
    pltpu.sync_copy(mb0, max_out.at[wid])
    pltpu.sync_copy(mb1, max_out.at[NW + wid])

  return _sddmm


NPS = 50176
RPS = NPS // NS
NPASS = 8
DP = D // NPASS


@functools.cache
def _make_agg(TPT, VT):
    _mesh = plsc.VectorSubcoreMesh(core_axis_name="c", subcore_axis_name="s")
    NBLK = TPT // 128

    @functools.partial(
        pl.kernel,
        out_type=(
            jax.ShapeDtypeStruct((NPASS * NC * NPS, DP), jnp.float32),
            jax.ShapeDtypeStruct((NC * NPS,), jnp.float32),
        ),
        mesh=_mesh,
        compiler_params=pltpu.CompilerParams(use_tc_tiling_on_sc=False),
        scratch_types=[
            pltpu.VMEM((TPT,), jnp.int32),
            pltpu.VMEM((TPT,), jnp.int32),
            pltpu.VMEM((TPT,), jnp.float32),
            pltpu.VMEM((128,), jnp.int32),
            pltpu.VMEM((128,), jnp.int32),
            pltpu.VMEM((128,), jnp.float32),
            pltpu.VMEM((128, DP), jnp.float32),
            pltpu.VMEM((128,), jnp.int32),
            pltpu.VMEM((128,), jnp.int32),
            pltpu.VMEM((128,), jnp.float32),
            pltpu.VMEM((128, DP), jnp.float32),
            pltpu.VMEM((L,), jnp.float32),
            pltpu.VMEM((392, DP), jnp.float32),
            pltpu.VMEM((392, DP), jnp.float32),
            pltpu.VMEM((RPS,), jnp.float32),
            pltpu.VMEM((RPS,), jnp.float32),
            pltpu.VMEM_SHARED((NPS, DP), jnp.float32),
            pltpu.VMEM_SHARED((NPS,), jnp.float32),
            pltpu.SemaphoreType.DMA,
            pltpu.SemaphoreType.DMA,
        ],
    )
    def _agg(vcat4, srcv, dstv, lgtv, mvec, agg_out, den_out,
             src_sh, dst_sh, lgt_sh, srcblkA, idxblkA, exblkA, rowsA,
             srcblkB, idxblkB, exblkB, rowsB, mv,
             zrows, stg, zden, stgd, sp_agg, sp_den, semA, semB):
        c = lax.axis_index("c")
        s = lax.axis_index("s")
        wid = s * NC + c
        base = wid * TPT
        z16 = jnp.zeros((L,), jnp.float32)

        pltpu.sync_copy(srcv.at[pl.ds(base, TPT)], src_sh)
        pltpu.sync_copy(dstv.at[pl.ds(base, TPT)], dst_sh)
        pltpu.sync_copy(lgtv.at[pl.ds(base, TPT)], lgt_sh)
        pltpu.sync_copy(mvec, mv)
        M = mv[...]

        def exf(i, _):
            sl = pl.ds(i * 16, 16)
            lgt_sh[sl] = jnp.exp(lgt_sh[sl] - M)
            return 0

        lax.fori_loop(0, TPT // 16, exf, 0)

        def zfill_rows(i, _):
            for j in range(DP // 16):
                zrows[i, pl.ds(j * 16, 16)] = z16
            return 0

        lax.fori_loop(0, 392, zfill_rows, 0)

        def zfill_den(i, _):
            zden[pl.ds(i * 16, 16)] = z16
            return 0

        lax.fori_loop(0, RPS // 16, zfill_den, 0)

        def build(boff, poff, srcblk, idxblk, exblk):
            for g in range(8):
                sl = pl.ds(boff + g * 16, 16)
                gsl = pl.ds(g * 16, 16)
                idxblk[gsl] = dst_sh[sl]
                srcblk[gsl] = src_sh[sl] + poff
                exblk[gsl] = lgt_sh[sl]

        def scale_store(p, srcblk, idxblk, exblk, rows, sem):
            pltpu.make_async_copy(vcat4.at[srcblk], rows, sem).wait()
            for g in range(8):
                ex = exblk[pl.ds(g * 16, 16)]
                for e in range(16):
                    r = g * 16 + e
                    exs = ex[e]
                    for j in range(DP // 16):
                        jsl = pl.ds(j * 16, 16)
                        rows[r, jsl] = rows[r, jsl] * exs
            pltpu.sync_copy(rows, sp_agg.at[idxblk], add=True)

            @pl.when(p == 0)
            def _():
                pltpu.sync_copy(exblk, sp_den.at[idxblk], add=True)

        def pass_body(p, _p):
            poff = p * VT
            for r in range(8):
                pltpu.sync_copy(zrows, sp_agg.at[pl.ds(s * RPS + r * 392, 392)])

            @pl.when(p == 0)
            def _():
                pltpu.sync_copy(zden, sp_den.at[pl.ds(s * RPS, RPS)])

            plsc.subcore_barrier()

            build(0, poff, srcblkA, idxblkA, exblkA)
            pltpu.async_copy(vcat4.at[srcblkA], rowsA, semA)

            def blk(i, _b):
                boff0 = 2 * i * 128
                build(boff0 + 128, poff, srcblkB, idxblkB, exblkB)
                pltpu.async_copy(vcat4.at[srcblkB], rowsB, semB)
                scale_store(p, srcblkA, idxblkA, exblkA, rowsA, semA)

                @pl.when(boff0 + 256 < TPT)
                def _():
                    build(boff0 + 256, poff, srcblkA, idxblkA, exblkA)
                    pltpu.async_copy(vcat4.at[srcblkA], rowsA, semA)

                scale_store(p, srcblkB, idxblkB, exblkB, rowsB, semB)
                return 0

            lax.fori_loop(0, NBLK // 2, blk, 0)
            plsc.subcore_barrier()
            obase = p * (NC * NPS) + c * NPS + s * RPS
            for r in range(8):
                pltpu.sync_copy(sp_agg.at[pl.ds(s * RPS + r * 392, 392)], stg)
                pltpu.sync_copy(stg, agg_out.at[pl.ds(obase + r * 392, 392)])

            @pl.when(p == 0)
            def _():
                pltpu.sync_copy(sp_den.at[pl.ds(s * RPS, RPS)], stgd)
                pltpu.sync_copy(stgd, den_out.at[pl.ds(c * NPS + s * RPS, RPS)])

            plsc.subcore_barrier()
            return 0

        lax.fori_loop(0, NPASS, pass_body, 0)

    return _agg


TPT_P = 12544
TPT_A = 6400
EPP = TPT_P * NW
EAP = TPT_A * NW


@functools.cache
def _build_pool():
  _mesh = plsc.VectorSubcoreMesh(core_axis_name="c", subcore_axis_name="s")

  @functools.partial(
    pl.kernel,
    out_type=jax.ShapeDtypeStruct((NC * B, D), jnp.float32),
    mesh=_mesh,
    scratch_types=[
        pltpu.VMEM((PBLK, D), jnp.float32),
        pltpu.VMEM((PBLK, D), jnp.float32),
        pltpu.VMEM((PBLK,), jnp.int32),
        pltpu.VMEM((PBLK,), jnp.int32),
        pltpu.VMEM((B, D), jnp.float32),
        pltpu.VMEM_SHARED((B, D), jnp.float32),
        pltpu.SemaphoreType.DMA,
        pltpu.SemaphoreType.DMA,
    ],
  )
  def _pool(x, cbi, out, rbufA, rbufB, cbufA, cbufB, zb, sp_pool, semA, semB):
    c = lax.axis_index("c")
    s = lax.axis_index("s")
    wid = s * NC + c
    base = wid * RPT
    z16 = jnp.zeros((L,), jnp.float32)

    def zf(i, _):
        for j in range(D // 16):
            zb[i, pl.ds(j * 16, 16)] = z16
        return 0

    lax.fori_loop(0, B, zf, 0)

    @pl.when(s == 0)
    def _():
        pltpu.sync_copy(zb, sp_pool)

    plsc.subcore_barrier()

    def issue(off, rbuf, cbuf, sem):
        pltpu.async_copy(x.at[pl.ds(off, PBLK)], rbuf, sem)
        pltpu.async_copy(cbi.at[pl.ds(off, PBLK)], cbuf, sem)

    def drain_add(off, rbuf, cbuf, sem):
        pltpu.make_async_copy(x.at[pl.ds(off, PBLK)], rbuf, sem).wait()
        pltpu.make_async_copy(cbi.at[pl.ds(off, PBLK)], cbuf, sem).wait()
        pltpu.sync_copy(rbuf, sp_pool.at[cbuf], add=True)

    issue(base, rbufA, cbufA, semA)

    def blk(i, _):
        off0 = base + 2 * i * PBLK
        issue(off0 + PBLK, rbufB, cbufB, semB)
        drain_add(off0, rbufA, cbufA, semA)

        @pl.when(2 * i + 2 < NPB)
        def _():
            issue(off0 + 2 * PBLK, rbufA, cbufA, semA)

        drain_add(off0 + PBLK, rbufB, cbufB, semB)
        return 0

    lax.fori_loop(0, NPB // 2, blk, 0)
    off_t = base + (NPB - 1) * PBLK
    drain_add(off_t, rbufA, cbufA, semA)
    plsc.subcore_barrier()

    @pl.when(s == 0)
    def _():
        pltpu.sync_copy(sp_pool, zb)
        pltpu.sync_copy(zb, out.at[pl.ds(c * B, B)])

  return _pool


def _lin(x, wb):
    return x @ wb[0] + wb[1]


def _hgt_layer_sc(xd, edges, lp):
    sq = jnp.sqrt(jnp.float32(D))
    kk = {t: _lin(xd[t], lp['k'][t]) for t in ('paper', 'author')}
    qq = {t: _lin(xd[t], lp['q'][t]) for t in ('paper', 'author')}
    vv = {t: _lin(xd[t], lp['v'][t]) for t in ('paper', 'author')}

    k_c = (kk['paper'] @ lp['a_rel']['cites']) * (lp['p_rel']['cites'] / sq)
    k_w = (kk['author'] @ lp['a_rel']['writes']) * (lp['p_rel']['writes'] / sq)
    k_r = (kk['paper'] @ lp['a_rel']['rev_writes']) * (lp['p_rel']['rev_writes'] / sq)
    v_c = vv['paper'] @ lp['m_rel']['cites']
    v_w = vv['author'] @ lp['m_rel']['writes']
    v_r = vv['paper'] @ lp['m_rel']['rev_writes']

    qcat = jnp.concatenate([qq['paper'], qq['author']])
    kcat = jnp.concatenate([k_c, k_w, k_r])

    src_c, dst_c = edges['cites'][0], edges['cites'][1]
    src_w, dst_w = edges['writes'][0], edges['writes'][1]
    src_r, dst_r = edges['rev_writes'][0], edges['rev_writes'][1]

    srck = _pad1(jnp.concatenate([src_c, src_w + N, src_r + 2 * N]), E3P, 0)
    dstq = _pad1(jnp.concatenate([dst_c, dst_w, dst_r + N]), E3P, 0)

    lgt, maxes = _build_sddmm()(qcat, kcat, srck, dstq)
    m_p = jnp.max(maxes[:NW])
    m_a = jnp.max(maxes[NW:])

    vcat_p = jnp.concatenate([v_c, v_w])
    vcat4_p = vcat_p.reshape(2 * N, NPASS, DP).transpose(1, 0, 2).reshape(-1, DP)
    srcv_p = _pad1(jnp.concatenate([src_c, src_w + N]), EPP, 0)
    dstv_p = _pad1(jnp.concatenate([dst_c, dst_w]), EPP, 0)
    lgt_p = _pad1(lgt[:2 * E], EPP, NEG)
    agg_p2, den_p2 = _make_agg(TPT_P, 2 * N)(vcat4_p, srcv_p, dstv_p, lgt_p,
                                             jnp.full((L,), m_p, jnp.float32))

    vcat4_a = v_r.reshape(N, NPASS, DP).transpose(1, 0, 2).reshape(-1, DP)
    srcv_a = _pad1(src_r, EAP, 0)
    dstv_a = _pad1(dst_r, EAP, 0)
    lgt_a = _pad1(lgt[2 * E:3 * E], EAP, NEG)
    agg_a2, den_a2 = _make_agg(TPT_A, N)(vcat4_a, srcv_a, dstv_a, lgt_a,
                                         jnp.full((L,), m_a, jnp.float32))

    out = {}
    for t, agg2, den2 in (('paper', agg_p2, den_p2), ('author', agg_a2, den_a2)):
        agg = (agg2.reshape(NPASS, NC, NPS, DP).sum(axis=1)[:, :N]
               .transpose(1, 0, 2).reshape(N, D))
        den = den2.reshape(NC, NPS).sum(axis=0)[:N]
        agg = agg / (den + 1e-16)[:, None]
        o = _lin(jax.nn.gelu(agg), lp['out'][t])
        sg = jax.nn.sigmoid(lp['skip'][t])
        out[t] = sg * o + (1.0 - sg) * xd[t]
    return out


def kernel(x_paper, x_author, edge_index_cites, edge_index_writes,
           edge_index_rev_writes, x_covs, batch_num, is_sparsed,
           class_batch_info, params):
    edges = {'cites': edge_index_cites, 'writes': edge_index_writes,
             'rev_writes': edge_index_rev_writes}
    xd = {
        'paper': _lin(x_paper, params['basis']['paper']),
        'author': _lin(x_author, params['basis']['author']),
    }
    for lp in params['layers']:
        xd = _hgt_layer_sc(xd, edges, lp)
        xd = {t: jax.nn.relu(v) for t, v in xd.items()}

    x = xd['paper']
    xpad = jnp.concatenate([x, jnp.zeros((NPOOL - N, D), jnp.float32)])
    cpad = _pad1(class_batch_info, NPOOL, 0)
    parts = _build_pool()(xpad, cpad)
    pooled = parts.reshape(NC, B, D).sum(axis=0)
    bounds = jnp.searchsorted(class_batch_info, jnp.arange(B + 1, dtype=jnp.int32))
    cnt = (bounds[1:] - bounds[:-1]).astype(jnp.float32)
    pooled = pooled / jnp.maximum(cnt, 1.0)[:, None]

    h = jnp.concatenate([pooled, x_covs], axis=1)
    h = jax.nn.relu(_lin(h, params['lin1']))
    return _lin(h, params['lin2'])
